# Initial kernel scaffold; baseline (speedup 1.0000x reference)
import functools

import jax
import jax.numpy as jnp
from jax import lax
from jax.experimental import pallas as pl
from jax.experimental.pallas import tpu as pltpu
from jax.experimental.pallas import tpu_sc as plsc

N = 10000
D = 128
E = 320000
DT = 1.0
ALPHA = 1.0
GAMMA = 1.0

NC = 2
NS = 16
NW = NC * NS
CHUNK = 128
EPT = -(-E // (NW * CHUNK)) * CHUNK
E_PAD = EPT * NW
NCHUNK = EPT // CHUNK
RPT = N // NS


def _sc_agg_body(src_hbm, dst_hbm, x_hbm, zero_hbm, out_hbm,
                 src_v, dst_v, rows_v, acc_sh, sem):
    c = lax.axis_index("c")
    s = lax.axis_index("s")
    wid = s * NC + c
    pltpu.sync_copy(zero_hbm, acc_sh.at[pl.ds(s * RPT, RPT)])
    pltpu.sync_copy(src_hbm.at[pl.ds(wid * NCHUNK, NCHUNK)], src_v)
    pltpu.sync_copy(dst_hbm.at[pl.ds(wid * NCHUNK, NCHUNK)], dst_v)
    plsc.subcore_barrier()

    def body(i, carry):
        pltpu.async_copy(x_hbm.at[src_v.at[i]], rows_v, sem).wait()
        pltpu.sync_copy(rows_v, acc_sh.at[dst_v.at[i]], add=True)
        return carry

    lax.fori_loop(0, NCHUNK, body, 0)
    plsc.subcore_barrier()
    pltpu.sync_copy(acc_sh.at[pl.ds(s * RPT, RPT)],
                    out_hbm.at[c, pl.ds(s * RPT, RPT)])


_sc_agg = functools.partial(
    pl.kernel,
    out_type=jax.ShapeDtypeStruct((NC, N, D), jnp.float32),
    mesh=plsc.VectorSubcoreMesh(core_axis_name="c", subcore_axis_name="s"),
    scratch_types=[
        pltpu.VMEM((NCHUNK, CHUNK), jnp.int32),
        pltpu.VMEM((NCHUNK, CHUNK), jnp.int32),
        pltpu.VMEM((CHUNK, D), jnp.float32),
        pltpu.VMEM_SHARED((N, D), jnp.float32),
        pltpu.SemaphoreType.DMA,
    ],
)(_sc_agg_body)


BLK = 1000


def _tc_update_body(p_ref, x_ref, y_ref, w_ref, b_ref, xo_ref, yo_ref):
    agg = p_ref[0] + p_ref[1]
    g = jnp.dot(agg, w_ref[...], preferred_element_type=jnp.float32) + b_ref[...]
    r = jnp.maximum(g, 0.0)
    x = x_ref[...]
    y = y_ref[...]
    ynew = y + DT * (r - ALPHA * y - GAMMA * x)
    xo_ref[...] = x + DT * ynew
    yo_ref[...] = ynew


def _tc_update(p, x, y, w, b):
    return pl.pallas_call(
        _tc_update_body,
        grid=(N // BLK,),
        in_specs=[
            pl.BlockSpec((NC, BLK, D), lambda i: (0, i, 0)),
            pl.BlockSpec((BLK, D), lambda i: (i, 0)),
            pl.BlockSpec((BLK, D), lambda i: (i, 0)),
            pl.BlockSpec((D, D), lambda i: (0, 0)),
            pl.BlockSpec((1, D), lambda i: (0, 0)),
        ],
        out_specs=[pl.BlockSpec((BLK, D), lambda i: (i, 0)),
                   pl.BlockSpec((BLK, D), lambda i: (i, 0))],
        out_shape=[jax.ShapeDtypeStruct((N, D), jnp.float32),
                   jax.ShapeDtypeStruct((N, D), jnp.float32)],
    )(p, x, y, w, b.reshape(1, D))


def kernel(X0, Y0, edge_index, W1, b1, W2, b2):
    src = edge_index[0].astype(jnp.int32)
    dst = edge_index[1].astype(jnp.int32)
    pad = E_PAD - E
    src = jnp.concatenate([src, jnp.full((pad,), N, jnp.int32)])
    dst = jnp.concatenate([dst, jnp.zeros((pad,), jnp.int32)])
    src = src.reshape(E_PAD // CHUNK, CHUNK)
    dst = dst.reshape(E_PAD // CHUNK, CHUNK)
    zero = jnp.zeros((RPT, D), jnp.float32)
    zrow = jnp.zeros((1, D), jnp.float32)

    x0_pad = jnp.concatenate([X0, zrow], axis=0)
    p1 = _sc_agg(src, dst, x0_pad, zero)
    X1, Y1 = _tc_update(p1, X0, Y0, W1, b1)

    x1_pad = jnp.concatenate([X1, zrow], axis=0)
    p2 = _sc_agg(src, dst, x1_pad, zero)
    X2, Y2 = _tc_update(p2, X1, Y1, W2, b2)
    return (X2, Y2)


# speedup vs baseline: 4.9423x; 4.9423x over previous
import functools

import jax
import jax.numpy as jnp
from jax import lax
from jax.experimental import pallas as pl
from jax.experimental.pallas import tpu as pltpu
from jax.experimental.pallas import tpu_sc as plsc

N = 10000
D = 128
E = 320000
DT = 1.0
ALPHA = 1.0
GAMMA = 1.0

NC = 2
NS = 16
NW = NC * NS
CHUNK = 128
EPT = -(-E // (NW * CHUNK)) * CHUNK
E_PAD = EPT * NW
NCHUNK = EPT // CHUNK
RPT = 632
N_PAD = RPT * NS


def _sc_agg_body(src_hbm, dst_hbm, x_hbm, zero_hbm, out_hbm,
                 src_v, dst_v, rows_v, acc_sh, sem):
    c = lax.axis_index("c")
    s = lax.axis_index("s")
    wid = s * NC + c
    row0 = pl.multiple_of(s * RPT, 8)
    pltpu.sync_copy(zero_hbm, acc_sh.at[pl.ds(row0, RPT)])
    pltpu.sync_copy(src_hbm.at[wid], src_v)
    pltpu.sync_copy(dst_hbm.at[wid], dst_v)
    plsc.subcore_barrier()

    def body(i, carry):
        pltpu.async_copy(x_hbm.at[src_v.at[i]], rows_v, sem).wait()
        pltpu.sync_copy(rows_v, acc_sh.at[dst_v.at[i]], add=True)
        return carry

    lax.fori_loop(0, NCHUNK, body, 0)
    plsc.subcore_barrier()
    pltpu.sync_copy(acc_sh.at[pl.ds(row0, RPT)],
                    out_hbm.at[c, pl.ds(row0, RPT)])


_sc_agg = functools.partial(
    pl.kernel,
    out_type=jax.ShapeDtypeStruct((NC, N_PAD, D), jnp.float32),
    mesh=plsc.VectorSubcoreMesh(core_axis_name="c", subcore_axis_name="s"),
    scratch_types=[
        pltpu.VMEM((NCHUNK, CHUNK), jnp.int32),
        pltpu.VMEM((NCHUNK, CHUNK), jnp.int32),
        pltpu.VMEM((CHUNK, D), jnp.float32),
        pltpu.VMEM_SHARED((N_PAD, D), jnp.float32),
        pltpu.SemaphoreType.DMA,
    ],
)(_sc_agg_body)


BLK = 1000


def _tc_update_body(p_ref, x_ref, y_ref, w_ref, b_ref, xo_ref, yo_ref):
    agg = p_ref[0] + p_ref[1]
    g = jnp.dot(agg, w_ref[...], preferred_element_type=jnp.float32) + b_ref[...]
    r = jnp.maximum(g, 0.0)
    x = x_ref[...]
    y = y_ref[...]
    ynew = y + DT * (r - ALPHA * y - GAMMA * x)
    xo_ref[...] = x + DT * ynew
    yo_ref[...] = ynew


def _tc_update(p, x, y, w, b):
    return pl.pallas_call(
        _tc_update_body,
        grid=(N // BLK,),
        in_specs=[
            pl.BlockSpec((NC, BLK, D), lambda i: (0, i, 0)),
            pl.BlockSpec((BLK, D), lambda i: (i, 0)),
            pl.BlockSpec((BLK, D), lambda i: (i, 0)),
            pl.BlockSpec((D, D), lambda i: (0, 0)),
            pl.BlockSpec((1, D), lambda i: (0, 0)),
        ],
        out_specs=[pl.BlockSpec((BLK, D), lambda i: (i, 0)),
                   pl.BlockSpec((BLK, D), lambda i: (i, 0))],
        out_shape=[jax.ShapeDtypeStruct((N, D), jnp.float32),
                   jax.ShapeDtypeStruct((N, D), jnp.float32)],
    )(p, x, y, w, b.reshape(1, D))


def kernel(X0, Y0, edge_index, W1, b1, W2, b2):
    src = edge_index[0].astype(jnp.int32)
    dst = edge_index[1].astype(jnp.int32)
    pad = E_PAD - E
    src = jnp.concatenate([src, jnp.full((pad,), N, jnp.int32)])
    dst = jnp.concatenate([dst, jnp.zeros((pad,), jnp.int32)])
    src = src.reshape(NW, NCHUNK, CHUNK)
    dst = dst.reshape(NW, NCHUNK, CHUNK)
    zero = jnp.zeros((RPT, D), jnp.float32)
    zrow = jnp.zeros((1, D), jnp.float32)

    x0_pad = jnp.concatenate([X0, zrow], axis=0)
    p1 = _sc_agg(src, dst, x0_pad, zero)
    X1, Y1 = _tc_update(p1, X0, Y0, W1, b1)

    x1_pad = jnp.concatenate([X1, zrow], axis=0)
    p2 = _sc_agg(src, dst, x1_pad, zero)
    X2, Y2 = _tc_update(p2, X1, Y1, W2, b2)
    return (X2, Y2)
